# Initial kernel scaffold; baseline (speedup 1.0000x reference)
import functools
import math

import numpy as np
import jax
import jax.numpy as jnp
from jax import lax
from jax.experimental import pallas as pl
from jax.experimental.pallas import tpu as pltpu
from jax.experimental.pallas import tpu_sc as plsc

C_IN = 4
D = 256
HEADS = 4
HD = D // HEADS
SUBDIV = 4
B, H, W = 2, 64, 120
N = H * W
BN = B * N

NS = 16
CPW = N // NS
CH = 120
NCH = CPW // CH
RB = 512


def _build_static():
    phi = (1 + 5 ** 0.5) / 2
    verts = np.array([(-1, phi, 0), (1, phi, 0), (-1, -phi, 0), (1, -phi, 0),
                      (0, -1, phi), (0, 1, phi), (0, -1, -phi), (0, 1, -phi),
                      (phi, 0, -1), (phi, 0, 1), (-phi, 0, -1), (-phi, 0, 1)],
                     dtype=np.float64)
    verts = verts / np.linalg.norm(verts, axis=1, keepdims=True)
    faces = np.array([(0, 11, 5), (0, 5, 1), (0, 1, 7), (0, 7, 10), (0, 10, 11),
                      (1, 5, 9), (5, 11, 4), (11, 10, 2), (10, 7, 6), (7, 1, 8),
                      (3, 9, 4), (3, 4, 2), (3, 2, 6), (3, 6, 8), (3, 8, 9),
                      (4, 9, 5), (2, 4, 11), (6, 2, 10), (8, 6, 7), (9, 8, 1)],
                     dtype=np.int64)
    verts = verts.astype(np.float32)

    for _ in range(SUBDIV):
        cache = {}
        new_faces = []
        vlist = verts.tolist()

        def midpoint(a, b):
            key = tuple(sorted((a, b)))
            if key in cache:
                return cache[key]
            va = np.array(vlist[a]); vb = np.array(vlist[b])
            vm = (va + vb) / 2.0
            vm = (vm / np.linalg.norm(vm)).tolist()
            vlist.append(vm)
            idx = len(vlist) - 1
            cache[key] = idx
            return idx

        for a, b, c in faces.tolist():
            ab = midpoint(a, b); bc = midpoint(b, c); ca = midpoint(c, a)
            new_faces.extend([(a, ab, ca), (b, bc, ab), (c, ca, bc), (ab, bc, ca)])
        nv = np.array(vlist, dtype=np.float32)
        verts = nv / np.linalg.norm(nv, axis=1, keepdims=True)
        faces = np.array(new_faces, dtype=np.int64)

    es = set()
    for f in faces.tolist():
        for i in range(3):
            a, b = f[i], f[(i + 1) % 3]
            es.add(tuple(sorted((a, b))))
    edges = np.array(sorted(es), dtype=np.int64)

    lat_c = np.linspace(-math.pi / 2, math.pi / 2, H, dtype=np.float32)
    lon_c = np.linspace(-math.pi, math.pi, W, dtype=np.float32)
    lon_g, lat_g = np.meshgrid(lon_c, lat_c, indexing='xy')
    flat = np.stack([lat_g, lon_g], axis=-1).reshape(-1, 2)
    vlat = np.arcsin(verts[:, 2])
    vlon = np.arctan2(verts[:, 1], verts[:, 0])
    dlat = flat[:, 0][None, :] - vlat[:, None]
    dlon = np.remainder(flat[:, 1][None, :] - vlon[:, None] + math.pi,
                        2 * math.pi) - math.pi
    dist2 = dlat ** 2 + dlon ** 2
    g2v = np.argmin(dist2, axis=0).reshape(W, H).transpose(1, 0).reshape(-1)

    v0 = verts[edges[:, 0]]; v1 = verts[edges[:, 1]]
    dot = np.clip(np.sum(v0 * v1, axis=1), -1.0, 1.0)
    length = np.arccos(dot)[:, None]
    lat0 = np.arcsin(v0[:, 2]); lon0 = np.arctan2(v0[:, 1], v0[:, 0])
    lat1 = np.arcsin(v1[:, 2]); lon1 = np.arctan2(v1[:, 1], v1[:, 0])
    dlat_e = (lat1 - lat0)[:, None]
    dlon_e = np.remainder((lon1 - lon0)[:, None] + math.pi, 2 * math.pi) - math.pi
    ef = np.concatenate([dlat_e, dlon_e, length], axis=1).astype(np.float32)

    src = edges[:, 0].astype(np.int32)
    dst = edges[:, 1].astype(np.int32)
    return g2v.astype(np.int32), src, dst, ef


_G2V, _SRC, _DST, _EF = _build_static()
_G2V2 = np.concatenate([_G2V, _G2V + N])
_SRC2 = np.concatenate([_SRC, _SRC + N])
_EF_PAD = np.concatenate([_EF, np.zeros((N, 5), np.float32)], axis=1)
_E4 = np.repeat(np.eye(HEADS, dtype=np.float32), HD, axis=1)
_E4T = _E4.T.copy()


def _linear_body(x_ref, w_ref, b_ref, o_ref):
    o_ref[...] = jnp.dot(x_ref[...], w_ref[...],
                         preferred_element_type=jnp.float32) + b_ref[...]


def _tc_linear(x, w, b, rb):
    m = x.shape[0]
    return pl.pallas_call(
        _linear_body,
        grid=(m // rb,),
        in_specs=[
            pl.BlockSpec((rb, x.shape[1]), lambda i: (i, 0)),
            pl.BlockSpec(w.shape, lambda i: (0, 0)),
            pl.BlockSpec((1, w.shape[1]), lambda i: (0, 0)),
        ],
        out_specs=pl.BlockSpec((rb, w.shape[1]), lambda i: (i, 0)),
        out_shape=jax.ShapeDtypeStruct((m, w.shape[1]), jnp.float32),
    )(x, w, b.reshape(1, -1))


def _edge_mlp_body(ef_ref, w1_ref, b1_ref, w2_ref, b2_ref, o_ref):
    h1 = jnp.maximum(
        jnp.dot(ef_ref[...], w1_ref[...], preferred_element_type=jnp.float32)
        + b1_ref[...], 0.0)
    o_ref[...] = jnp.dot(h1, w2_ref[...],
                         preferred_element_type=jnp.float32) + b2_ref[...]


def _tc_edge_mlp(ef, w1, b1, w2, b2):
    rb = 1920
    return pl.pallas_call(
        _edge_mlp_body,
        grid=(N // rb,),
        in_specs=[
            pl.BlockSpec((rb, 8), lambda i: (i, 0)),
            pl.BlockSpec((8, D), lambda i: (0, 0)),
            pl.BlockSpec((1, D), lambda i: (0, 0)),
            pl.BlockSpec((D, D), lambda i: (0, 0)),
            pl.BlockSpec((1, D), lambda i: (0, 0)),
        ],
        out_specs=pl.BlockSpec((rb, D), lambda i: (i, 0)),
        out_shape=jax.ShapeDtypeStruct((N, D), jnp.float32),
    )(ef, w1, b1.reshape(1, -1), w2, b2.reshape(1, -1))


def _qkvu_body(hn_ref, wq, bq, wk, bk, wv, bv, wl, bl,
               q_ref, k_ref, v_ref, u_ref):
    hn = hn_ref[...]
    q_ref[...] = jnp.dot(hn, wq[...], preferred_element_type=jnp.float32) + bq[...]
    k_ref[...] = jnp.dot(hn, wk[...], preferred_element_type=jnp.float32) + bk[...]
    v_ref[...] = jnp.dot(hn, wv[...], preferred_element_type=jnp.float32) + bv[...]
    u_ref[...] = jnp.dot(hn, wl[...], preferred_element_type=jnp.float32) + bl[...]


def _ln_qkvu_body(hp_ref, up_ref, ag_ref, g_ref, b2_ref,
                  wq, bq, wk, bk, wv, bv, wl, bl,
                  hn_ref, q_ref, k_ref, v_ref, u_ref):
    tmp = up_ref[...] + ag_ref[...]
    mu = jnp.mean(tmp, axis=-1, keepdims=True)
    var = jnp.mean((tmp - mu) ** 2, axis=-1, keepdims=True)
    hn = hp_ref[...] + (tmp - mu) / jnp.sqrt(var + 1e-5) * g_ref[...] + b2_ref[...]
    hn_ref[...] = hn
    q_ref[...] = jnp.dot(hn, wq[...], preferred_element_type=jnp.float32) + bq[...]
    k_ref[...] = jnp.dot(hn, wk[...], preferred_element_type=jnp.float32) + bk[...]
    v_ref[...] = jnp.dot(hn, wv[...], preferred_element_type=jnp.float32) + bv[...]
    u_ref[...] = jnp.dot(hn, wl[...], preferred_element_type=jnp.float32) + bl[...]


def _w_spec():
    return pl.BlockSpec((D, D), lambda i: (0, 0))


def _b_spec():
    return pl.BlockSpec((1, D), lambda i: (0, 0))


def _r_spec():
    return pl.BlockSpec((RB, D), lambda i: (i, 0))


def _tc_qkvu(hn, wq, bq, wk, bk, wv, bv, wl, bl):
    os = jax.ShapeDtypeStruct((BN, D), jnp.float32)
    return pl.pallas_call(
        _qkvu_body,
        grid=(BN // RB,),
        in_specs=[_r_spec(),
                  _w_spec(), _b_spec(), _w_spec(), _b_spec(),
                  _w_spec(), _b_spec(), _w_spec(), _b_spec()],
        out_specs=[_r_spec()] * 4,
        out_shape=[os] * 4,
    )(hn, wq, bq.reshape(1, -1), wk, bk.reshape(1, -1),
      wv, bv.reshape(1, -1), wl, bl.reshape(1, -1))


def _tc_ln_qkvu(hp, up, ag, g, b2, wq, bq, wk, bk, wv, bv, wl, bl):
    os = jax.ShapeDtypeStruct((BN, D), jnp.float32)
    return pl.pallas_call(
        _ln_qkvu_body,
        grid=(BN // RB,),
        in_specs=[_r_spec(), _r_spec(), _r_spec(), _b_spec(), _b_spec(),
                  _w_spec(), _b_spec(), _w_spec(), _b_spec(),
                  _w_spec(), _b_spec(), _w_spec(), _b_spec()],
        out_specs=[_r_spec()] * 5,
        out_shape=[os] * 5,
    )(hp, up, ag, g.reshape(1, -1), b2.reshape(1, -1),
      wq, bq.reshape(1, -1), wk, bk.reshape(1, -1),
      wv, bv.reshape(1, -1), wl, bl.reshape(1, -1))


def _kq_body(k_ref, qs_ref, es_ref, e4t_ref, ap_ref):
    kq = k_ref[...] * (qs_ref[...] + es_ref[...])
    ap_ref[...] = jnp.dot(kq, e4t_ref[...],
                          preferred_element_type=jnp.float32) * (1.0 / 16.0)


def _tc_kq(k2, qsum, esum, e4t):
    nrb = N // RB
    return pl.pallas_call(
        _kq_body,
        grid=(BN // RB,),
        in_specs=[_r_spec(), _r_spec(),
                  pl.BlockSpec((RB, D), lambda i: (i % nrb, 0)),
                  pl.BlockSpec((D, HEADS), lambda i: (0, 0))],
        out_specs=pl.BlockSpec((RB, HEADS), lambda i: (i, 0)),
        out_shape=jax.ShapeDtypeStruct((BN, HEADS), jnp.float32),
    )(k2, qsum, esum, e4t)


def _softmax_body(ap_ref, al_ref):
    ap = ap_ref[...]
    m = jnp.max(ap, axis=1, keepdims=True)
    e = jnp.exp(ap - m)
    al_ref[...] = e / jnp.sum(e, axis=1, keepdims=True)


def _tc_softmax(ap):
    return pl.pallas_call(
        _softmax_body,
        grid=(B,),
        in_specs=[pl.BlockSpec((1, N, HEADS), lambda i: (i, 0, 0))],
        out_specs=pl.BlockSpec((1, N, HEADS), lambda i: (i, 0, 0)),
        out_shape=jax.ShapeDtypeStruct((B, N, HEADS), jnp.float32),
    )(ap.reshape(B, N, HEADS))


def _msg_body(al_ref, e4_ref, vs_ref, ee_ref, o_ref):
    wexp = jnp.dot(al_ref[...], e4_ref[...], preferred_element_type=jnp.float32)
    o_ref[...] = wexp * (vs_ref[...] + ee_ref[...])


def _tc_msg(alpha, e4, v_src, e_emb):
    nrb = N // RB
    return pl.pallas_call(
        _msg_body,
        grid=(BN // RB,),
        in_specs=[pl.BlockSpec((RB, HEADS), lambda i: (i, 0)),
                  pl.BlockSpec((HEADS, D), lambda i: (0, 0)),
                  _r_spec(),
                  pl.BlockSpec((RB, D), lambda i: (i % nrb, 0))],
        out_specs=_r_spec(),
        out_shape=jax.ShapeDtypeStruct((BN, D), jnp.float32),
    )(alpha.reshape(BN, HEADS), e4, v_src, e_emb)


def _final_body(hp_ref, up_ref, ag_ref, g_ref, b2_ref, wo_ref, bo_ref, y_ref):
    tmp = up_ref[...] + ag_ref[...]
    mu = jnp.mean(tmp, axis=-1, keepdims=True)
    var = jnp.mean((tmp - mu) ** 2, axis=-1, keepdims=True)
    hn = hp_ref[...] + (tmp - mu) / jnp.sqrt(var + 1e-5) * g_ref[...] + b2_ref[...]
    y_ref[...] = jnp.dot(hn, wo_ref[...],
                         preferred_element_type=jnp.float32) + bo_ref[...]


def _tc_final(hp, up, ag, g, b2, wo_pad, bo_pad):
    return pl.pallas_call(
        _final_body,
        grid=(BN // RB,),
        in_specs=[_r_spec(), _r_spec(), _r_spec(), _b_spec(), _b_spec(),
                  pl.BlockSpec((D, 16), lambda i: (0, 0)),
                  pl.BlockSpec((1, 16), lambda i: (0, 0))],
        out_specs=pl.BlockSpec((RB, 16), lambda i: (i, 0)),
        out_shape=jax.ShapeDtypeStruct((BN, 16), jnp.float32),
    )(hp, up, ag, g.reshape(1, -1), b2.reshape(1, -1), wo_pad, bo_pad)


def _sc_mesh():
    return plsc.VectorSubcoreMesh(core_axis_name="c", subcore_axis_name="s",
                                  num_cores=2, num_subcores=NS)


def _zero_rows(rows_v):
    z16 = jnp.zeros((16,), jnp.float32)
    nlane = rows_v.shape[1] // 16

    def body(i, carry):
        for j in range(nlane):
            rows_v[i, pl.ds(j * 16, 16)] = z16
        return carry

    lax.fori_loop(0, rows_v.shape[0], body, 0)


def _sc_prep_body(z_hbm, g2v2_hbm, ee_hbm, dst_hbm,
                  hn0_hbm, esum_hbm, idx_v, rows_v, acc_sh, sem):
    cid = lax.axis_index("c")
    sid = lax.axis_index("s")
    base = sid * CPW
    for j in range(NCH):
        off = cid * N + base + j * CH
        pltpu.sync_copy(g2v2_hbm.at[pl.ds(off, CH)], idx_v)
        pltpu.async_copy(z_hbm.at[idx_v], rows_v, sem).wait()
        pltpu.sync_copy(rows_v, hn0_hbm.at[pl.ds(off, CH)])

    @pl.when(cid == 0)
    def _():
        _zero_rows(rows_v)
        for j in range(NCH):
            pltpu.sync_copy(rows_v, acc_sh.at[pl.ds(base + j * CH, CH)])
        plsc.subcore_barrier()
        for j in range(NCH):
            off = base + j * CH
            pltpu.sync_copy(dst_hbm.at[pl.ds(off, CH)], idx_v)
            pltpu.sync_copy(ee_hbm.at[pl.ds(off, CH)], rows_v)
            pltpu.sync_copy(rows_v, acc_sh.at[idx_v], add=True)
        plsc.subcore_barrier()
        pltpu.sync_copy(acc_sh.at[pl.ds(base, CPW)], esum_hbm.at[pl.ds(base, CPW)])


def _sc_prep(z, g2v2, e_emb, dst):
    f = pl.kernel(
        _sc_prep_body,
        out_type=[jax.ShapeDtypeStruct((BN, D), jnp.float32),
                  jax.ShapeDtypeStruct((N, D), jnp.float32)],
        mesh=_sc_mesh(),
        scratch_types=[pltpu.VMEM((CH,), jnp.int32),
                       pltpu.VMEM((CH, D), jnp.float32),
                       pltpu.VMEM_SHARED((N, D), jnp.float32),
                       pltpu.SemaphoreType.DMA],
    )
    return f(z, g2v2, e_emb, dst)


def _sc_qv_body(q_hbm, v_hbm, src2_hbm, dst_hbm,
                qsum_hbm, vsrc_hbm, idx_v, idxd_v, rows_v, acc_sh, sem):
    cid = lax.axis_index("c")
    sid = lax.axis_index("s")
    base = sid * CPW
    _zero_rows(rows_v)
    for j in range(NCH):
        pltpu.sync_copy(rows_v, acc_sh.at[pl.ds(base + j * CH, CH)])
    plsc.subcore_barrier()
    for j in range(NCH):
        goff = cid * N + base + j * CH
        loff = base + j * CH
        pltpu.sync_copy(src2_hbm.at[pl.ds(goff, CH)], idx_v)
        pltpu.async_copy(q_hbm.at[idx_v], rows_v, sem).wait()
        pltpu.sync_copy(dst_hbm.at[pl.ds(loff, CH)], idxd_v)
        pltpu.sync_copy(rows_v, acc_sh.at[idxd_v], add=True)
        pltpu.async_copy(v_hbm.at[idx_v], rows_v, sem).wait()
        pltpu.sync_copy(rows_v, vsrc_hbm.at[pl.ds(goff, CH)])
    plsc.subcore_barrier()
    pltpu.sync_copy(acc_sh.at[pl.ds(base, CPW)],
                    qsum_hbm.at[pl.ds(cid * N + base, CPW)])


def _sc_qv(q2, v2, src2, dst):
    f = pl.kernel(
        _sc_qv_body,
        out_type=[jax.ShapeDtypeStruct((BN, D), jnp.float32),
                  jax.ShapeDtypeStruct((BN, D), jnp.float32)],
        mesh=_sc_mesh(),
        scratch_types=[pltpu.VMEM((CH,), jnp.int32),
                       pltpu.VMEM((CH,), jnp.int32),
                       pltpu.VMEM((CH, D), jnp.float32),
                       pltpu.VMEM_SHARED((N, D), jnp.float32),
                       pltpu.SemaphoreType.DMA],
    )
    return f(q2, v2, src2, dst)


def _sc_agg_body(msg_hbm, dst_hbm, agg_hbm, idxd_v, rows_v, acc_sh):
    cid = lax.axis_index("c")
    sid = lax.axis_index("s")
    base = sid * CPW
    _zero_rows(rows_v)
    for j in range(NCH):
        pltpu.sync_copy(rows_v, acc_sh.at[pl.ds(base + j * CH, CH)])
    plsc.subcore_barrier()
    for j in range(NCH):
        goff = cid * N + base + j * CH
        loff = base + j * CH
        pltpu.sync_copy(msg_hbm.at[pl.ds(goff, CH)], rows_v)
        pltpu.sync_copy(dst_hbm.at[pl.ds(loff, CH)], idxd_v)
        pltpu.sync_copy(rows_v, acc_sh.at[idxd_v], add=True)
    plsc.subcore_barrier()
    pltpu.sync_copy(acc_sh.at[pl.ds(base, CPW)],
                    agg_hbm.at[pl.ds(cid * N + base, CPW)])


def _sc_agg(msg, dst):
    f = pl.kernel(
        _sc_agg_body,
        out_type=jax.ShapeDtypeStruct((BN, D), jnp.float32),
        mesh=_sc_mesh(),
        scratch_types=[pltpu.VMEM((CH,), jnp.int32),
                       pltpu.VMEM((CH, D), jnp.float32),
                       pltpu.VMEM_SHARED((N, D), jnp.float32)],
    )
    return f(msg, dst)


def _sc_out_body(y_hbm, g2v2_hbm, og_hbm, idx_v, rows_v, sem):
    cid = lax.axis_index("c")
    sid = lax.axis_index("s")
    base = sid * CPW
    for j in range(NCH):
        off = cid * N + base + j * CH
        pltpu.sync_copy(g2v2_hbm.at[pl.ds(off, CH)], idx_v)
        pltpu.async_copy(y_hbm.at[idx_v], rows_v, sem).wait()
        pltpu.sync_copy(rows_v, og_hbm.at[pl.ds(off, CH)])


def _sc_out(ypad, g2v2):
    f = pl.kernel(
        _sc_out_body,
        out_type=jax.ShapeDtypeStruct((BN, 16), jnp.float32),
        mesh=_sc_mesh(),
        scratch_types=[pltpu.VMEM((CH,), jnp.int32),
                       pltpu.VMEM((CH, 16), jnp.float32),
                       pltpu.SemaphoreType.DMA],
    )
    return f(ypad, g2v2)


def kernel(x, t, Wi, bi, Wq, bq, Wk, bk, Wv, bv, We1, be1, We2, be2,
           Wl0, bl0, Wl1, bl1, Wl2, bl2, Wl3, bl3, Wo, bo, ln_g, ln_b):
    f32 = jnp.float32
    g2v2 = jnp.asarray(_G2V2)
    src2 = jnp.asarray(_SRC2)
    dst = jnp.asarray(_DST)
    ef_pad = jnp.asarray(_EF_PAD)
    e4 = jnp.asarray(_E4)
    e4t = jnp.asarray(_E4T)

    nodes = jnp.transpose(x, (0, 2, 3, 1)).reshape(BN, C_IN)
    nodes_pad = jnp.concatenate(
        [nodes, jnp.zeros((BN, 8 - C_IN), f32)], axis=1)
    wi_pad = jnp.concatenate([Wi, jnp.zeros((8 - C_IN, D), f32)], axis=0)
    we1_pad = jnp.concatenate([We1, jnp.zeros((5, D), f32)], axis=0)
    wo_pad = jnp.concatenate([Wo, jnp.zeros((D, 12), f32)], axis=1)
    bo_pad = jnp.concatenate([bo, jnp.zeros((12,), f32)]).reshape(1, 16)

    z = _tc_linear(nodes_pad, wi_pad, bi, 1920)
    e_emb = _tc_edge_mlp(ef_pad, we1_pad, be1, We2, be2)
    hn, esum = _sc_prep(z, g2v2, e_emb, dst)

    up = agg = None
    for li, (wl, bl) in enumerate(((Wl0, bl0), (Wl1, bl1), (Wl2, bl2), (Wl3, bl3))):
        if li == 0:
            q2, k2, v2, up_new = _tc_qkvu(hn, Wq, bq, Wk, bk, Wv, bv, wl, bl)
        else:
            hn, q2, k2, v2, up_new = _tc_ln_qkvu(
                hn, up, agg, ln_g, ln_b, Wq, bq, Wk, bk, Wv, bv, wl, bl)
        up = up_new
        qsum, v_src = _sc_qv(q2, v2, src2, dst)
        ap = _tc_kq(k2, qsum, esum, e4t)
        alpha = _tc_softmax(ap)
        msg = _tc_msg(alpha, e4, v_src, e_emb)
        agg = _sc_agg(msg, dst)

    ypad = _tc_final(hn, up, agg, ln_g, ln_b, wo_pad, bo_pad)
    og = _sc_out(ypad, g2v2)
    out = og.reshape(B, N, 16)[:, :, :C_IN].reshape(B, H, W, C_IN)
    return jnp.transpose(out, (0, 3, 1, 2))


# speedup vs baseline: 7.8537x; 7.8537x over previous
import math

import numpy as np
import jax
import jax.numpy as jnp
from jax import lax
from jax.experimental import pallas as pl
from jax.experimental.pallas import tpu as pltpu
from jax.experimental.pallas import tpu_sc as plsc

C_IN = 4
D = 256
DH = D // 2
HEADS = 4
HD = D // HEADS
SUBDIV = 4
B, H, W = 2, 64, 120
N = H * W
BN = B * N

NS = 16
CPW = N // NS
CH = 120
NCH = CPW // CH
RB = 512


def _build_static():
    phi = (1 + 5 ** 0.5) / 2
    verts = np.array([(-1, phi, 0), (1, phi, 0), (-1, -phi, 0), (1, -phi, 0),
                      (0, -1, phi), (0, 1, phi), (0, -1, -phi), (0, 1, -phi),
                      (phi, 0, -1), (phi, 0, 1), (-phi, 0, -1), (-phi, 0, 1)],
                     dtype=np.float64)
    verts = verts / np.linalg.norm(verts, axis=1, keepdims=True)
    faces = np.array([(0, 11, 5), (0, 5, 1), (0, 1, 7), (0, 7, 10), (0, 10, 11),
                      (1, 5, 9), (5, 11, 4), (11, 10, 2), (10, 7, 6), (7, 1, 8),
                      (3, 9, 4), (3, 4, 2), (3, 2, 6), (3, 6, 8), (3, 8, 9),
                      (4, 9, 5), (2, 4, 11), (6, 2, 10), (8, 6, 7), (9, 8, 1)],
                     dtype=np.int64)
    verts = verts.astype(np.float32)

    for _ in range(SUBDIV):
        cache = {}
        new_faces = []
        vlist = verts.tolist()

        def midpoint(a, b):
            key = tuple(sorted((a, b)))
            if key in cache:
                return cache[key]
            va = np.array(vlist[a]); vb = np.array(vlist[b])
            vm = (va + vb) / 2.0
            vm = (vm / np.linalg.norm(vm)).tolist()
            vlist.append(vm)
            idx = len(vlist) - 1
            cache[key] = idx
            return idx

        for a, b, c in faces.tolist():
            ab = midpoint(a, b); bc = midpoint(b, c); ca = midpoint(c, a)
            new_faces.extend([(a, ab, ca), (b, bc, ab), (c, ca, bc), (ab, bc, ca)])
        nv = np.array(vlist, dtype=np.float32)
        verts = nv / np.linalg.norm(nv, axis=1, keepdims=True)
        faces = np.array(new_faces, dtype=np.int64)

    es = set()
    for f in faces.tolist():
        for i in range(3):
            a, b = f[i], f[(i + 1) % 3]
            es.add(tuple(sorted((a, b))))
    edges = np.array(sorted(es), dtype=np.int64)

    lat_c = np.linspace(-math.pi / 2, math.pi / 2, H, dtype=np.float32)
    lon_c = np.linspace(-math.pi, math.pi, W, dtype=np.float32)
    lon_g, lat_g = np.meshgrid(lon_c, lat_c, indexing='xy')
    flat = np.stack([lat_g, lon_g], axis=-1).reshape(-1, 2)
    vlat = np.arcsin(verts[:, 2])
    vlon = np.arctan2(verts[:, 1], verts[:, 0])
    dlat = flat[:, 0][None, :] - vlat[:, None]
    dlon = np.remainder(flat[:, 1][None, :] - vlon[:, None] + math.pi,
                        2 * math.pi) - math.pi
    dist2 = dlat ** 2 + dlon ** 2
    g2v = np.argmin(dist2, axis=0).reshape(W, H).transpose(1, 0).reshape(-1)

    v0 = verts[edges[:, 0]]; v1 = verts[edges[:, 1]]
    dot = np.clip(np.sum(v0 * v1, axis=1), -1.0, 1.0)
    length = np.arccos(dot)[:, None]
    lat0 = np.arcsin(v0[:, 2]); lon0 = np.arctan2(v0[:, 1], v0[:, 0])
    lat1 = np.arcsin(v1[:, 2]); lon1 = np.arctan2(v1[:, 1], v1[:, 0])
    dlat_e = (lat1 - lat0)[:, None]
    dlon_e = np.remainder((lon1 - lon0)[:, None] + math.pi, 2 * math.pi) - math.pi
    ef = np.concatenate([dlat_e, dlon_e, length], axis=1).astype(np.float32)

    src = edges[:, 0].astype(np.int32)
    dst = edges[:, 1].astype(np.int32)
    return g2v.astype(np.int32), src, dst, ef


_G2V, _SRC, _DST, _EF = _build_static()
_G2V2 = np.concatenate([_G2V, _G2V + N])
_SRC2 = np.concatenate([_SRC, _SRC + N])
_EF_PAD = np.concatenate([_EF, np.zeros((N, 5), np.float32)], axis=1)
_E4 = np.repeat(np.eye(HEADS, dtype=np.float32), HD, axis=1)
_E4T = _E4.T.copy()


def _linear_body(x_ref, w_ref, b_ref, o_ref):
    o_ref[...] = jnp.dot(x_ref[...], w_ref[...],
                         preferred_element_type=jnp.float32) + b_ref[...]


def _tc_linear(x, w, b, rb):
    m = x.shape[0]
    return pl.pallas_call(
        _linear_body,
        grid=(m // rb,),
        in_specs=[
            pl.BlockSpec((rb, x.shape[1]), lambda i: (i, 0)),
            pl.BlockSpec(w.shape, lambda i: (0, 0)),
            pl.BlockSpec((1, w.shape[1]), lambda i: (0, 0)),
        ],
        out_specs=pl.BlockSpec((rb, w.shape[1]), lambda i: (i, 0)),
        out_shape=jax.ShapeDtypeStruct((m, w.shape[1]), jnp.float32),
    )(x, w, b.reshape(1, -1))


def _edge_mlp_body(ef_ref, w1_ref, b1_ref, w2_ref, b2_ref,
                   o_ref, ol_ref, or_ref):
    h1 = jnp.maximum(
        jnp.dot(ef_ref[...], w1_ref[...], preferred_element_type=jnp.float32)
        + b1_ref[...], 0.0)
    ee = jnp.dot(h1, w2_ref[...], preferred_element_type=jnp.float32) + b2_ref[...]
    o_ref[...] = ee
    ol_ref[...] = ee[:, :DH]
    or_ref[...] = ee[:, DH:]


def _tc_edge_mlp(ef, w1, b1, w2, b2):
    rb = 1920
    return pl.pallas_call(
        _edge_mlp_body,
        grid=(N // rb,),
        in_specs=[
            pl.BlockSpec((rb, 8), lambda i: (i, 0)),
            pl.BlockSpec((8, D), lambda i: (0, 0)),
            pl.BlockSpec((1, D), lambda i: (0, 0)),
            pl.BlockSpec((D, D), lambda i: (0, 0)),
            pl.BlockSpec((1, D), lambda i: (0, 0)),
        ],
        out_specs=[pl.BlockSpec((rb, D), lambda i: (i, 0)),
                   pl.BlockSpec((rb, DH), lambda i: (i, 0)),
                   pl.BlockSpec((rb, DH), lambda i: (i, 0))],
        out_shape=[jax.ShapeDtypeStruct((N, D), jnp.float32),
                   jax.ShapeDtypeStruct((N, DH), jnp.float32),
                   jax.ShapeDtypeStruct((N, DH), jnp.float32)],
    )(ef, w1, b1.reshape(1, -1), w2, b2.reshape(1, -1))


def _w_spec():
    return pl.BlockSpec((D, D), lambda i: (0, 0))


def _b_spec():
    return pl.BlockSpec((1, D), lambda i: (0, 0))


def _r_spec():
    return pl.BlockSpec((RB, D), lambda i: (i, 0))


def _h_spec():
    return pl.BlockSpec((RB, DH), lambda i: (i, 0))


def _qkvu_common(hn, wq, bq, wk, bk, wv, bv, wl, bl,
                 ql_ref, qr_ref, k_ref, v_ref, u_ref):
    q = jnp.dot(hn, wq[...], preferred_element_type=jnp.float32) + bq[...]
    ql_ref[...] = q[:, :DH]
    qr_ref[...] = q[:, DH:]
    k_ref[...] = jnp.dot(hn, wk[...], preferred_element_type=jnp.float32) + bk[...]
    v_ref[...] = jnp.dot(hn, wv[...], preferred_element_type=jnp.float32) + bv[...]
    u_ref[...] = jnp.dot(hn, wl[...], preferred_element_type=jnp.float32) + bl[...]


def _qkvu_body(hn_ref, wq, bq, wk, bk, wv, bv, wl, bl,
               ql_ref, qr_ref, k_ref, v_ref, u_ref):
    _qkvu_common(hn_ref[...], wq, bq, wk, bk, wv, bv, wl, bl,
                 ql_ref, qr_ref, k_ref, v_ref, u_ref)


def _ln_qkvu_body(hp_ref, up_ref, agl_ref, agr_ref, g_ref, b2_ref,
                  wq, bq, wk, bk, wv, bv, wl, bl,
                  hn_ref, ql_ref, qr_ref, k_ref, v_ref, u_ref):
    agg = jnp.concatenate([agl_ref[...], agr_ref[...]], axis=1)
    tmp = up_ref[...] + agg
    mu = jnp.mean(tmp, axis=-1, keepdims=True)
    var = jnp.mean((tmp - mu) ** 2, axis=-1, keepdims=True)
    hn = hp_ref[...] + (tmp - mu) / jnp.sqrt(var + 1e-5) * g_ref[...] + b2_ref[...]
    hn_ref[...] = hn
    _qkvu_common(hn, wq, bq, wk, bk, wv, bv, wl, bl,
                 ql_ref, qr_ref, k_ref, v_ref, u_ref)


def _tc_qkvu(hn, wq, bq, wk, bk, wv, bv, wl, bl):
    osd = jax.ShapeDtypeStruct((BN, D), jnp.float32)
    osh = jax.ShapeDtypeStruct((BN, DH), jnp.float32)
    return pl.pallas_call(
        _qkvu_body,
        grid=(BN // RB,),
        in_specs=[_r_spec(),
                  _w_spec(), _b_spec(), _w_spec(), _b_spec(),
                  _w_spec(), _b_spec(), _w_spec(), _b_spec()],
        out_specs=[_h_spec(), _h_spec(), _r_spec(), _r_spec(), _r_spec()],
        out_shape=[osh, osh, osd, osd, osd],
    )(hn, wq, bq.reshape(1, -1), wk, bk.reshape(1, -1),
      wv, bv.reshape(1, -1), wl, bl.reshape(1, -1))


def _tc_ln_qkvu(hp, up, aggl, aggr, g, b2, wq, bq, wk, bk, wv, bv, wl, bl):
    osd = jax.ShapeDtypeStruct((BN, D), jnp.float32)
    osh = jax.ShapeDtypeStruct((BN, DH), jnp.float32)
    return pl.pallas_call(
        _ln_qkvu_body,
        grid=(BN // RB,),
        in_specs=[_r_spec(), _r_spec(), _h_spec(), _h_spec(),
                  _b_spec(), _b_spec(),
                  _w_spec(), _b_spec(), _w_spec(), _b_spec(),
                  _w_spec(), _b_spec(), _w_spec(), _b_spec()],
        out_specs=[_r_spec(), _h_spec(), _h_spec(), _r_spec(), _r_spec(),
                   _r_spec()],
        out_shape=[osd, osh, osh, osd, osd, osd],
    )(hp, up, aggl, aggr, g.reshape(1, -1), b2.reshape(1, -1),
      wq, bq.reshape(1, -1), wk, bk.reshape(1, -1),
      wv, bv.reshape(1, -1), wl, bl.reshape(1, -1))


def _kq_body(k_ref, qsl_ref, qsr_ref, esl_ref, esr_ref, e4t_ref, ap_ref):
    k = k_ref[...]
    kql = k[:, :DH] * (qsl_ref[...] + esl_ref[...])
    kqr = k[:, DH:] * (qsr_ref[...] + esr_ref[...])
    e4t = e4t_ref[...]
    ap = (jnp.dot(kql, e4t[:DH], preferred_element_type=jnp.float32)
          + jnp.dot(kqr, e4t[DH:], preferred_element_type=jnp.float32))
    ap_ref[...] = ap * (1.0 / 16.0)


def _tc_kq(k2, qsuml, qsumr, esuml, esumr, e4t):
    nrb = N // RB
    return pl.pallas_call(
        _kq_body,
        grid=(BN // RB,),
        in_specs=[_r_spec(), _h_spec(), _h_spec(),
                  pl.BlockSpec((RB, DH), lambda i: (i % nrb, 0)),
                  pl.BlockSpec((RB, DH), lambda i: (i % nrb, 0)),
                  pl.BlockSpec((D, HEADS), lambda i: (0, 0))],
        out_specs=pl.BlockSpec((RB, HEADS), lambda i: (i, 0)),
        out_shape=jax.ShapeDtypeStruct((BN, HEADS), jnp.float32),
    )(k2, qsuml, qsumr, esuml, esumr, e4t)


def _softmax_body(ap_ref, al_ref):
    ap = ap_ref[...]
    m = jnp.max(ap, axis=1, keepdims=True)
    e = jnp.exp(ap - m)
    al_ref[...] = e / jnp.sum(e, axis=1, keepdims=True)


def _tc_softmax(ap):
    return pl.pallas_call(
        _softmax_body,
        grid=(B,),
        in_specs=[pl.BlockSpec((1, N, HEADS), lambda i: (i, 0, 0))],
        out_specs=pl.BlockSpec((1, N, HEADS), lambda i: (i, 0, 0)),
        out_shape=jax.ShapeDtypeStruct((B, N, HEADS), jnp.float32),
    )(ap.reshape(B, N, HEADS))


def _msg_body(al_ref, e4_ref, vs_ref, ee_ref, ol_ref, or_ref):
    wexp = jnp.dot(al_ref[...], e4_ref[...], preferred_element_type=jnp.float32)
    msg = wexp * (vs_ref[...] + ee_ref[...])
    ol_ref[...] = msg[:, :DH]
    or_ref[...] = msg[:, DH:]


def _tc_msg(alpha, e4, v_src, e_emb):
    nrb = N // RB
    osh = jax.ShapeDtypeStruct((BN, DH), jnp.float32)
    return pl.pallas_call(
        _msg_body,
        grid=(BN // RB,),
        in_specs=[pl.BlockSpec((RB, HEADS), lambda i: (i, 0)),
                  pl.BlockSpec((HEADS, D), lambda i: (0, 0)),
                  _r_spec(),
                  pl.BlockSpec((RB, D), lambda i: (i % nrb, 0))],
        out_specs=[_h_spec(), _h_spec()],
        out_shape=[osh, osh],
    )(alpha.reshape(BN, HEADS), e4, v_src, e_emb)


def _final_body(hp_ref, up_ref, agl_ref, agr_ref, g_ref, b2_ref,
                wo_ref, bo_ref, y_ref):
    agg = jnp.concatenate([agl_ref[...], agr_ref[...]], axis=1)
    tmp = up_ref[...] + agg
    mu = jnp.mean(tmp, axis=-1, keepdims=True)
    var = jnp.mean((tmp - mu) ** 2, axis=-1, keepdims=True)
    hn = hp_ref[...] + (tmp - mu) / jnp.sqrt(var + 1e-5) * g_ref[...] + b2_ref[...]
    y_ref[...] = jnp.dot(hn, wo_ref[...],
                         preferred_element_type=jnp.float32) + bo_ref[...]


def _tc_final(hp, up, aggl, aggr, g, b2, wo_pad, bo_pad):
    return pl.pallas_call(
        _final_body,
        grid=(BN // RB,),
        in_specs=[_r_spec(), _r_spec(), _h_spec(), _h_spec(),
                  _b_spec(), _b_spec(),
                  pl.BlockSpec((D, DH), lambda i: (0, 0)),
                  pl.BlockSpec((1, DH), lambda i: (0, 0))],
        out_specs=pl.BlockSpec((RB, DH), lambda i: (i, 0)),
        out_shape=jax.ShapeDtypeStruct((BN, DH), jnp.float32),
    )(hp, up, aggl, aggr, g.reshape(1, -1), b2.reshape(1, -1), wo_pad, bo_pad)


def _sc_mesh():
    return plsc.VectorSubcoreMesh(core_axis_name="c", subcore_axis_name="s",
                                  num_cores=2, num_subcores=NS)


def _zero_rows(rows_v):
    z16 = jnp.zeros((16,), jnp.float32)
    nlane = rows_v.shape[1] // 16

    def body(i, carry):
        for j in range(nlane):
            rows_v[i, pl.ds(j * 16, 16)] = z16
        return carry

    lax.fori_loop(0, rows_v.shape[0], body, 0)


def _sc_prep_body(z_hbm, g2v2_hbm, eel_hbm, eer_hbm, dst_hbm,
                  hn0_hbm, esuml_hbm, esumr_hbm,
                  idx_v, rows_v, hrows_v, acc_sh, sem):
    cid = lax.axis_index("c")
    sid = lax.axis_index("s")
    base = sid * CPW
    for j in range(NCH):
        off = cid * N + base + j * CH
        pltpu.sync_copy(g2v2_hbm.at[pl.ds(off, CH)], idx_v)
        pltpu.async_copy(z_hbm.at[idx_v], rows_v, sem).wait()
        pltpu.sync_copy(rows_v, hn0_hbm.at[pl.ds(off, CH)])

    @pl.when(cid == 0)
    def _():
        _zero_rows(hrows_v)
        for half, (ee_hbm, es_hbm) in enumerate(((eel_hbm, esuml_hbm),
                                                 (eer_hbm, esumr_hbm))):
            for j in range(NCH):
                pltpu.sync_copy(hrows_v, acc_sh.at[pl.ds(base + j * CH, CH)])
            plsc.subcore_barrier()
            for j in range(NCH):
                off = base + j * CH
                pltpu.sync_copy(dst_hbm.at[pl.ds(off, CH)], idx_v)
                pltpu.sync_copy(ee_hbm.at[pl.ds(off, CH)], hrows_v)
                pltpu.sync_copy(hrows_v, acc_sh.at[idx_v], add=True)
            plsc.subcore_barrier()
            pltpu.sync_copy(acc_sh.at[pl.ds(base, CPW)],
                            es_hbm.at[pl.ds(base, CPW)])
            if half == 0:
                _zero_rows(hrows_v)
                plsc.subcore_barrier()


def _sc_prep(z, g2v2, eel, eer, dst):
    f = pl.kernel(
        _sc_prep_body,
        out_type=[jax.ShapeDtypeStruct((BN, D), jnp.float32),
                  jax.ShapeDtypeStruct((N, DH), jnp.float32),
                  jax.ShapeDtypeStruct((N, DH), jnp.float32)],
        mesh=_sc_mesh(),
        scratch_types=[pltpu.VMEM((CH,), jnp.int32),
                       pltpu.VMEM((CH, D), jnp.float32),
                       pltpu.VMEM((CH, DH), jnp.float32),
                       pltpu.VMEM_SHARED((N, DH), jnp.float32),
                       pltpu.SemaphoreType.DMA],
    )
    return f(z, g2v2, eel, eer, dst)


def _sc_qv_body(ql_hbm, qr_hbm, v_hbm, src2_hbm, dst_hbm,
                qsuml_hbm, qsumr_hbm, vsrc_hbm,
                idx_v, idxd_v, rows_v, hrows_v, acc_sh, sem):
    cid = lax.axis_index("c")
    sid = lax.axis_index("s")
    base = sid * CPW
    _zero_rows(hrows_v)
    for half, (q_hbm, qs_hbm) in enumerate(((ql_hbm, qsuml_hbm),
                                            (qr_hbm, qsumr_hbm))):
        for j in range(NCH):
            pltpu.sync_copy(hrows_v, acc_sh.at[pl.ds(base + j * CH, CH)])
        plsc.subcore_barrier()
        for j in range(NCH):
            goff = cid * N + base + j * CH
            loff = base + j * CH
            pltpu.sync_copy(src2_hbm.at[pl.ds(goff, CH)], idx_v)
            pltpu.async_copy(q_hbm.at[idx_v], hrows_v, sem).wait()
            pltpu.sync_copy(dst_hbm.at[pl.ds(loff, CH)], idxd_v)
            pltpu.sync_copy(hrows_v, acc_sh.at[idxd_v], add=True)
        plsc.subcore_barrier()
        pltpu.sync_copy(acc_sh.at[pl.ds(base, CPW)],
                        qs_hbm.at[pl.ds(cid * N + base, CPW)])
        if half == 0:
            _zero_rows(hrows_v)
            plsc.subcore_barrier()
    for j in range(NCH):
        goff = cid * N + base + j * CH
        pltpu.sync_copy(src2_hbm.at[pl.ds(goff, CH)], idx_v)
        pltpu.async_copy(v_hbm.at[idx_v], rows_v, sem).wait()
        pltpu.sync_copy(rows_v, vsrc_hbm.at[pl.ds(goff, CH)])


def _sc_qv(ql, qr, v2, src2, dst):
    osh = jax.ShapeDtypeStruct((BN, DH), jnp.float32)
    f = pl.kernel(
        _sc_qv_body,
        out_type=[osh, osh, jax.ShapeDtypeStruct((BN, D), jnp.float32)],
        mesh=_sc_mesh(),
        scratch_types=[pltpu.VMEM((CH,), jnp.int32),
                       pltpu.VMEM((CH,), jnp.int32),
                       pltpu.VMEM((CH, D), jnp.float32),
                       pltpu.VMEM((CH, DH), jnp.float32),
                       pltpu.VMEM_SHARED((N, DH), jnp.float32),
                       pltpu.SemaphoreType.DMA],
    )
    return f(ql, qr, v2, src2, dst)


def _sc_agg_body(msgl_hbm, msgr_hbm, dst_hbm, aggl_hbm, aggr_hbm,
                 idxd_v, hrows_v, acc_sh):
    cid = lax.axis_index("c")
    sid = lax.axis_index("s")
    base = sid * CPW
    _zero_rows(hrows_v)
    for half, (msg_hbm, agg_hbm) in enumerate(((msgl_hbm, aggl_hbm),
                                               (msgr_hbm, aggr_hbm))):
        for j in range(NCH):
            pltpu.sync_copy(hrows_v, acc_sh.at[pl.ds(base + j * CH, CH)])
        plsc.subcore_barrier()
        for j in range(NCH):
            goff = cid * N + base + j * CH
            loff = base + j * CH
            pltpu.sync_copy(msg_hbm.at[pl.ds(goff, CH)], hrows_v)
            pltpu.sync_copy(dst_hbm.at[pl.ds(loff, CH)], idxd_v)
            pltpu.sync_copy(hrows_v, acc_sh.at[idxd_v], add=True)
        plsc.subcore_barrier()
        pltpu.sync_copy(acc_sh.at[pl.ds(base, CPW)],
                        agg_hbm.at[pl.ds(cid * N + base, CPW)])
        if half == 0:
            _zero_rows(hrows_v)
            plsc.subcore_barrier()


def _sc_agg(msgl, msgr, dst):
    osh = jax.ShapeDtypeStruct((BN, DH), jnp.float32)
    f = pl.kernel(
        _sc_agg_body,
        out_type=[osh, osh],
        mesh=_sc_mesh(),
        scratch_types=[pltpu.VMEM((CH,), jnp.int32),
                       pltpu.VMEM((CH, DH), jnp.float32),
                       pltpu.VMEM_SHARED((N, DH), jnp.float32)],
    )
    return f(msgl, msgr, dst)


def _sc_out_body(y_hbm, g2v2_hbm, og_hbm, idx_v, rows_v, sem):
    cid = lax.axis_index("c")
    sid = lax.axis_index("s")
    base = sid * CPW
    for j in range(NCH):
        off = cid * N + base + j * CH
        pltpu.sync_copy(g2v2_hbm.at[pl.ds(off, CH)], idx_v)
        pltpu.async_copy(y_hbm.at[idx_v], rows_v, sem).wait()
        pltpu.sync_copy(rows_v, og_hbm.at[pl.ds(off, CH)])


def _sc_out(ypad, g2v2):
    f = pl.kernel(
        _sc_out_body,
        out_type=jax.ShapeDtypeStruct((BN, DH), jnp.float32),
        mesh=_sc_mesh(),
        scratch_types=[pltpu.VMEM((CH,), jnp.int32),
                       pltpu.VMEM((CH, DH), jnp.float32),
                       pltpu.SemaphoreType.DMA],
    )
    return f(ypad, g2v2)


def kernel(x, t, Wi, bi, Wq, bq, Wk, bk, Wv, bv, We1, be1, We2, be2,
           Wl0, bl0, Wl1, bl1, Wl2, bl2, Wl3, bl3, Wo, bo, ln_g, ln_b):
    f32 = jnp.float32
    g2v2 = jnp.asarray(_G2V2)
    src2 = jnp.asarray(_SRC2)
    dst = jnp.asarray(_DST)
    ef_pad = jnp.asarray(_EF_PAD)
    e4 = jnp.asarray(_E4)
    e4t = jnp.asarray(_E4T)

    nodes = jnp.transpose(x, (0, 2, 3, 1)).reshape(BN, C_IN)
    nodes_pad = jnp.concatenate(
        [nodes, jnp.zeros((BN, 8 - C_IN), f32)], axis=1)
    wi_pad = jnp.concatenate([Wi, jnp.zeros((8 - C_IN, D), f32)], axis=0)
    we1_pad = jnp.concatenate([We1, jnp.zeros((5, D), f32)], axis=0)
    wo_pad = jnp.concatenate([Wo, jnp.zeros((D, DH - C_IN), f32)], axis=1)
    bo_pad = jnp.concatenate([bo, jnp.zeros((DH - C_IN,), f32)]).reshape(1, DH)

    z = _tc_linear(nodes_pad, wi_pad, bi, 1920)
    e_emb, eel, eer = _tc_edge_mlp(ef_pad, we1_pad, be1, We2, be2)
    hn, esuml, esumr = _sc_prep(z, g2v2, eel, eer, dst)

    up = aggl = aggr = None
    for li, (wl, bl) in enumerate(((Wl0, bl0), (Wl1, bl1), (Wl2, bl2), (Wl3, bl3))):
        if li == 0:
            ql, qr, k2, v2, up_new = _tc_qkvu(hn, Wq, bq, Wk, bk, Wv, bv, wl, bl)
        else:
            hn, ql, qr, k2, v2, up_new = _tc_ln_qkvu(
                hn, up, aggl, aggr, ln_g, ln_b, Wq, bq, Wk, bk, Wv, bv, wl, bl)
        up = up_new
        qsuml, qsumr, v_src = _sc_qv(ql, qr, v2, src2, dst)
        ap = _tc_kq(k2, qsuml, qsumr, esuml, esumr, e4t)
        alpha = _tc_softmax(ap)
        msgl, msgr = _tc_msg(alpha, e4, v_src, e_emb)
        aggl, aggr = _sc_agg(msgl, msgr, dst)

    ypad = _tc_final(hn, up, aggl, aggr, ln_g, ln_b, wo_pad, bo_pad)
    og = _sc_out(ypad, g2v2)
    out = og.reshape(B, N, DH)[:, :, :C_IN].reshape(B, H, W, C_IN)
    return jnp.transpose(out, (0, 3, 1, 2))
